# baseline (speedup 1.0000x reference)
import functools

import jax
import jax.numpy as jnp
from jax import lax
from jax.experimental import pallas as pl
from jax.experimental.pallas import tpu as pltpu
from jax.experimental.pallas import tpu_sc as plsc

N = 10000
E = 320000
D = 128

NC = 2
NS = 16
NW = NC * NS
CH = 128

NP = 10240
ROWS_PER_TILE = NP // NS
K_PER_W = 80
NCHUNKS = K_PER_W * NW
E_PAD = NCHUNKS * CH
NBUF = 4


def _mesh():
    return plsc.VectorSubcoreMesh(core_axis_name="c", subcore_axis_name="s")


def _deg_call(dstc, zeros1d, ones1d):
    @functools.partial(
        pl.kernel,
        out_type=jax.ShapeDtypeStruct((NC, NP), jnp.float32),
        mesh=_mesh(),
        scratch_types=[
            pltpu.VMEM((K_PER_W, CH), jnp.int32),
            pltpu.VMEM((CH,), jnp.float32),
            pltpu.VMEM_SHARED((NP,), jnp.float32),
        ],
    )
    def k(dst_hbm, z_hbm, ones_hbm, out_hbm, dst_v, ones_v, acc):
        c = lax.axis_index("c")
        s = lax.axis_index("s")
        wid = c * NS + s
        pltpu.sync_copy(z_hbm.at[pl.ds(s * ROWS_PER_TILE, ROWS_PER_TILE)],
                        acc.at[pl.ds(s * ROWS_PER_TILE, ROWS_PER_TILE)])
        pltpu.sync_copy(dst_hbm.at[pl.ds(wid * K_PER_W, K_PER_W)], dst_v)
        pltpu.sync_copy(ones_hbm, ones_v)
        plsc.subcore_barrier()

        def body(j, carry):
            pltpu.sync_copy(ones_v, acc.at[dst_v.at[j]], add=True)
            return carry

        lax.fori_loop(0, K_PER_W, body, 0)
        plsc.subcore_barrier()
        pltpu.sync_copy(acc.at[pl.ds(s * ROWS_PER_TILE, ROWS_PER_TILE)],
                        out_hbm.at[c, pl.ds(s * ROWS_PER_TILE, ROWS_PER_TILE)])

    return k(dstc, zeros1d, ones1d)


def _prop_impl(y, srcc, dstc, zeros2d, hh, col_split):
    KT = NCHUNKS // NS if col_split else K_PER_W
    LAG = NBUF // 2

    @functools.partial(
        pl.kernel,
        out_type=jax.ShapeDtypeStruct((NC, NP, hh), jnp.float32),
        mesh=_mesh(),
        scratch_types=(
            [pltpu.VMEM_SHARED((NP, hh), jnp.float32),
             pltpu.VMEM((KT, CH), jnp.int32),
             pltpu.VMEM((KT, CH), jnp.int32)]
            + [pltpu.VMEM((CH, hh), jnp.float32) for _ in range(NBUF)]
            + [pltpu.SemaphoreType.DMA for _ in range(NBUF)]
            + [pltpu.SemaphoreType.DMA for _ in range(NBUF)]
        ),
        compiler_params=pltpu.CompilerParams(use_tc_tiling_on_sc=False),
    )
    def k(y_hbm, src_hbm, dst_hbm, z_hbm, out_hbm, acc, src_v, dst_v, *rest):
        rows = rest[:NBUF]
        semg = rest[NBUF:2 * NBUF]
        sems = rest[2 * NBUF:]
        c = lax.axis_index("c")
        s = lax.axis_index("s")
        r0 = s * ROWS_PER_TILE
        ysrc = y_hbm.at[c] if col_split else y_hbm
        base = s * KT if col_split else (c * NS + s) * KT
        pltpu.sync_copy(z_hbm.at[pl.ds(r0, ROWS_PER_TILE)],
                        acc.at[pl.ds(r0, ROWS_PER_TILE)])
        pltpu.sync_copy(src_hbm.at[pl.ds(base, KT)], src_v)
        pltpu.sync_copy(dst_hbm.at[pl.ds(base, KT)], dst_v)
        plsc.subcore_barrier()

        def gather(j, b):
            pltpu.async_copy(ysrc.at[src_v.at[j]], rows[b], semg[b])

        def wait_gather(j, b):
            pltpu.make_async_copy(ysrc.at[src_v.at[j]], rows[b], semg[b]).wait()

        def scatter(j, b):
            pltpu.async_copy(rows[b], acc.at[dst_v.at[j]], sems[b], add=True)

        def wait_scatter(j, b):
            pltpu.make_async_copy(rows[b], acc.at[dst_v.at[j]], sems[b]).wait()

        for b in range(NBUF):
            gather(b, b)

        def body(jo, carry):
            for b in range(NBUF):
                j = jo * NBUF + b
                wait_gather(j, b)
                scatter(j, b)
                n = j + LAG
                bn = (b + LAG) % NBUF

                @pl.when(jnp.logical_and(n >= NBUF, n < KT))
                def _():
                    wait_scatter(n - NBUF, bn)
                    gather(n, bn)
            return carry

        lax.fori_loop(0, KT // NBUF, body, 0)
        for b in range(NBUF):
            wait_scatter(KT - NBUF + b, (KT - NBUF + b) % NBUF)
        plsc.subcore_barrier()
        pltpu.sync_copy(acc.at[pl.ds(r0, ROWS_PER_TILE)],
                        out_hbm.at[c, pl.ds(r0, ROWS_PER_TILE)])

    return k(y, srcc, dstc, zeros2d)


_TC_GRID_BN = 2048


def _tc_pre(x, w1, d0, d1):
    def body(x_ref, w_ref, d0_ref, d1_ref, dinv_ref, y_ref):
        dinv = lax.rsqrt(d0_ref[...] + d1_ref[...] + 1.0)
        dinv_ref[...] = dinv
        xw = dinv * jnp.dot(x_ref[...], w_ref[...],
                            preferred_element_type=jnp.float32)
        y_ref[0] = xw[:, :64]
        y_ref[1] = xw[:, 64:]

    bn = _TC_GRID_BN
    return pl.pallas_call(
        body,
        grid=(NP // bn,),
        in_specs=[
            pl.BlockSpec((bn, D), lambda i: (i, 0)),
            pl.BlockSpec((D, D), lambda i: (0, 0)),
            pl.BlockSpec((bn, 1), lambda i: (i, 0)),
            pl.BlockSpec((bn, 1), lambda i: (i, 0)),
        ],
        out_specs=[
            pl.BlockSpec((bn, 1), lambda i: (i, 0)),
            pl.BlockSpec((NC, bn, 64), lambda i: (0, i, 0)),
        ],
        out_shape=[
            jax.ShapeDtypeStruct((NP, 1), jnp.float32),
            jax.ShapeDtypeStruct((NC, NP, 64), jnp.float32),
        ],
    )(x, w1, d0, d1)


def _tc_mid1(p, y1t, dinv, b1t, w2t):
    def body(p_ref, y_ref, dinv_ref, b_ref, w_ref, yout_ref):
        dinv = dinv_ref[...]
        s = dinv[None] * (p_ref[...] + y_ref[...]) + b_ref[...]
        hmid = jnp.where(s >= 0, s, 0.2 * s)
        acc = (jnp.dot(hmid[0], w_ref[0], preferred_element_type=jnp.float32)
               + jnp.dot(hmid[1], w_ref[1], preferred_element_type=jnp.float32))
        yout_ref[...] = dinv * acc

    bn = _TC_GRID_BN
    hout = w2t.shape[-1]
    return pl.pallas_call(
        body,
        grid=(NP // bn,),
        in_specs=[
            pl.BlockSpec((NC, bn, 64), lambda i: (0, i, 0)),
            pl.BlockSpec((NC, bn, 64), lambda i: (0, i, 0)),
            pl.BlockSpec((bn, 1), lambda i: (i, 0)),
            pl.BlockSpec((NC, 1, 64), lambda i: (0, 0, 0)),
            pl.BlockSpec((NC, 64, hout), lambda i: (0, 0, 0)),
        ],
        out_specs=pl.BlockSpec((bn, hout), lambda i: (i, 0)),
        out_shape=jax.ShapeDtypeStruct((NP, hout), jnp.float32),
    )(p, y1t, dinv, b1t, w2t)


def _tc_mid(p0, p1, y, dinv, b, w, h, hout):
    def body(p0_ref, p1_ref, y_ref, dinv_ref, b_ref, w_ref, yout_ref):
        s = dinv_ref[...] * (p0_ref[...] + p1_ref[...] + y_ref[...]) + b_ref[...]
        hmid = jnp.where(s >= 0, s, 0.2 * s)
        yout_ref[...] = dinv_ref[...] * jnp.dot(hmid, w_ref[...],
                                                preferred_element_type=jnp.float32)

    bn = _TC_GRID_BN
    return pl.pallas_call(
        body,
        grid=(NP // bn,),
        in_specs=[
            pl.BlockSpec((bn, h), lambda i: (i, 0)),
            pl.BlockSpec((bn, h), lambda i: (i, 0)),
            pl.BlockSpec((bn, h), lambda i: (i, 0)),
            pl.BlockSpec((bn, 1), lambda i: (i, 0)),
            pl.BlockSpec((1, h), lambda i: (0, 0)),
            pl.BlockSpec((h, hout), lambda i: (0, 0)),
        ],
        out_specs=pl.BlockSpec((bn, hout), lambda i: (i, 0)),
        out_shape=jax.ShapeDtypeStruct((NP, hout), jnp.float32),
    )(p0, p1, y, dinv, b, w)


def _tc_final(p0, p1, y, dinv, b, h):
    def body(p0_ref, p1_ref, y_ref, dinv_ref, b_ref, out_ref):
        out_ref[...] = (dinv_ref[...] * (p0_ref[...] + p1_ref[...] + y_ref[...])
                        + b_ref[...])

    bn = _TC_GRID_BN
    return pl.pallas_call(
        body,
        grid=(NP // bn,),
        in_specs=[
            pl.BlockSpec((bn, h), lambda i: (i, 0)),
            pl.BlockSpec((bn, h), lambda i: (i, 0)),
            pl.BlockSpec((bn, h), lambda i: (i, 0)),
            pl.BlockSpec((bn, 1), lambda i: (i, 0)),
            pl.BlockSpec((1, h), lambda i: (0, 0)),
        ],
        out_specs=pl.BlockSpec((bn, h), lambda i: (i, 0)),
        out_shape=jax.ShapeDtypeStruct((NP, h), jnp.float32),
    )(p0, p1, y, dinv, b)


def kernel(x, edge_index, W1, b1, W2, b2, W3, b3):
    H1 = W1.shape[1]
    H2 = W2.shape[1]
    C = W3.shape[1]

    src = edge_index[0]
    dst = edge_index[1]
    pad_e = E_PAD - E
    pad_idx = jnp.full((pad_e,), N, dtype=jnp.int32)
    srcc = jnp.concatenate([src, pad_idx]).reshape(NCHUNKS, CH)
    dstc = jnp.concatenate([dst, pad_idx]).reshape(NCHUNKS, CH)

    xp = jnp.zeros((NP, D), jnp.float32).at[:N].set(x)
    ones1d = jnp.ones((CH,), jnp.float32)
    zeros1d = jnp.zeros((NP,), jnp.float32)
    zeros64 = jnp.zeros((NP, 64), jnp.float32)
    zerosH2 = jnp.zeros((NP, H2), jnp.float32)
    zerosC = jnp.zeros((NP, C), jnp.float32)

    deg = _deg_call(dstc, zeros1d, ones1d)
    d0 = deg[0].reshape(NP, 1)
    d1 = deg[1].reshape(NP, 1)
    dinv, y1t = _tc_pre(xp, W1, d0, d1)

    pt = _prop_impl(y1t, srcc, dstc, zeros64, 64, True)
    y2 = _tc_mid1(pt, y1t, dinv, b1.reshape(NC, 1, 64), W2.reshape(NC, 64, H2))

    p = _prop_impl(y2, srcc, dstc, zerosH2, H2, False)
    y3 = _tc_mid(p[0], p[1], y2, dinv, b2.reshape(1, H2), W3, H2, C)

    p = _prop_impl(y3, srcc, dstc, zerosC, C, False)
    out = _tc_final(p[0], p[1], y3, dinv, b3.reshape(1, C), C)
    return out[:N]


# speedup vs baseline: 1.4394x; 1.4394x over previous
import functools

import jax
import jax.numpy as jnp
from jax import lax
from jax.experimental import pallas as pl
from jax.experimental.pallas import tpu as pltpu
from jax.experimental.pallas import tpu_sc as plsc

N = 10000
E = 320000
D = 128

NC = 2
NS = 16
NW = NC * NS
CH = 128

NP = 10240
ROWS_PER_TILE = NP // NS
K_PER_W = 80
NCHUNKS = K_PER_W * NW
E_PAD = NCHUNKS * CH
NBUF = 4


def _mesh():
    return plsc.VectorSubcoreMesh(core_axis_name="c", subcore_axis_name="s")


def _deg_call(dstc, zeros1d, ones1d):
    @functools.partial(
        pl.kernel,
        out_type=jax.ShapeDtypeStruct((NC, NP), jnp.float32),
        mesh=_mesh(),
        scratch_types=[
            pltpu.VMEM((K_PER_W, CH), jnp.int32),
            pltpu.VMEM((CH,), jnp.float32),
            pltpu.VMEM_SHARED((NP,), jnp.float32),
        ],
    )
    def k(dst_hbm, z_hbm, ones_hbm, out_hbm, dst_v, ones_v, acc):
        c = lax.axis_index("c")
        s = lax.axis_index("s")
        wid = c * NS + s
        pltpu.sync_copy(z_hbm.at[pl.ds(s * ROWS_PER_TILE, ROWS_PER_TILE)],
                        acc.at[pl.ds(s * ROWS_PER_TILE, ROWS_PER_TILE)])
        pltpu.sync_copy(dst_hbm.at[pl.ds(wid * K_PER_W, K_PER_W)], dst_v)
        pltpu.sync_copy(ones_hbm, ones_v)
        plsc.subcore_barrier()

        def body(j, carry):
            pltpu.sync_copy(ones_v, acc.at[dst_v.at[j]], add=True)
            return carry

        lax.fori_loop(0, K_PER_W, body, 0)
        plsc.subcore_barrier()
        pltpu.sync_copy(acc.at[pl.ds(s * ROWS_PER_TILE, ROWS_PER_TILE)],
                        out_hbm.at[c, pl.ds(s * ROWS_PER_TILE, ROWS_PER_TILE)])

    return k(dstc, zeros1d, ones1d)


def _prop_impl(y, srcc, dstc, zeros2d, hh, col_split, dt=jnp.float32):
    KT = NCHUNKS // NS if col_split else K_PER_W
    LAG = NBUF // 2

    @functools.partial(
        pl.kernel,
        out_type=jax.ShapeDtypeStruct((NC, NP, hh), dt),
        mesh=_mesh(),
        scratch_types=(
            [pltpu.VMEM_SHARED((NP, hh), dt),
             pltpu.VMEM((KT, CH), jnp.int32),
             pltpu.VMEM((KT, CH), jnp.int32)]
            + [pltpu.VMEM((CH, hh), dt) for _ in range(NBUF)]
            + [pltpu.SemaphoreType.DMA for _ in range(NBUF)]
            + [pltpu.SemaphoreType.DMA for _ in range(NBUF)]
        ),
        compiler_params=pltpu.CompilerParams(use_tc_tiling_on_sc=False),
    )
    def k(y_hbm, src_hbm, dst_hbm, z_hbm, out_hbm, acc, src_v, dst_v, *rest):
        rows = rest[:NBUF]
        semg = rest[NBUF:2 * NBUF]
        sems = rest[2 * NBUF:]
        c = lax.axis_index("c")
        s = lax.axis_index("s")
        r0 = s * ROWS_PER_TILE
        ysrc = y_hbm.at[c] if col_split else y_hbm
        base = s * KT if col_split else (c * NS + s) * KT
        pltpu.sync_copy(z_hbm.at[pl.ds(r0, ROWS_PER_TILE)],
                        acc.at[pl.ds(r0, ROWS_PER_TILE)])
        pltpu.sync_copy(src_hbm.at[pl.ds(base, KT)], src_v)
        pltpu.sync_copy(dst_hbm.at[pl.ds(base, KT)], dst_v)
        plsc.subcore_barrier()

        def gather(j, b):
            pltpu.async_copy(ysrc.at[src_v.at[j]], rows[b], semg[b])

        def wait_gather(j, b):
            pltpu.make_async_copy(ysrc.at[src_v.at[j]], rows[b], semg[b]).wait()

        def scatter(j, b):
            pltpu.async_copy(rows[b], acc.at[dst_v.at[j]], sems[b], add=True)

        def wait_scatter(j, b):
            pltpu.make_async_copy(rows[b], acc.at[dst_v.at[j]], sems[b]).wait()

        for b in range(NBUF):
            gather(b, b)

        def body(jo, carry):
            for b in range(NBUF):
                j = jo * NBUF + b
                wait_gather(j, b)
                scatter(j, b)
                n = j + LAG
                bn = (b + LAG) % NBUF

                @pl.when(jnp.logical_and(n >= NBUF, n < KT))
                def _():
                    wait_scatter(n - NBUF, bn)
                    gather(n, bn)
            return carry

        lax.fori_loop(0, KT // NBUF, body, 0)
        for b in range(NBUF):
            wait_scatter(KT - NBUF + b, (KT - NBUF + b) % NBUF)
        plsc.subcore_barrier()
        pltpu.sync_copy(acc.at[pl.ds(r0, ROWS_PER_TILE)],
                        out_hbm.at[c, pl.ds(r0, ROWS_PER_TILE)])

    return k(y, srcc, dstc, zeros2d)


_TC_GRID_BN = 2048


def _tc_pre(x, w1, d0, d1):
    def body(x_ref, w_ref, d0_ref, d1_ref, dinv_ref, y_ref):
        dinv = lax.rsqrt(d0_ref[...] + d1_ref[...] + 1.0)
        dinv_ref[...] = dinv
        xw = dinv * jnp.dot(x_ref[...], w_ref[...],
                            preferred_element_type=jnp.float32)
        y_ref[0] = xw[:, :64].astype(jnp.bfloat16)
        y_ref[1] = xw[:, 64:].astype(jnp.bfloat16)

    bn = _TC_GRID_BN
    return pl.pallas_call(
        body,
        grid=(NP // bn,),
        in_specs=[
            pl.BlockSpec((bn, D), lambda i: (i, 0)),
            pl.BlockSpec((D, D), lambda i: (0, 0)),
            pl.BlockSpec((bn, 1), lambda i: (i, 0)),
            pl.BlockSpec((bn, 1), lambda i: (i, 0)),
        ],
        out_specs=[
            pl.BlockSpec((bn, 1), lambda i: (i, 0)),
            pl.BlockSpec((NC, bn, 64), lambda i: (0, i, 0)),
        ],
        out_shape=[
            jax.ShapeDtypeStruct((NP, 1), jnp.float32),
            jax.ShapeDtypeStruct((NC, NP, 64), jnp.bfloat16),
        ],
    )(x, w1, d0, d1)


def _tc_mid1(p, y1t, dinv, b1t, w2t):
    def body(p_ref, y_ref, dinv_ref, b_ref, w_ref, yout_ref):
        dinv = dinv_ref[...]
        s = (dinv[None] * (p_ref[...].astype(jnp.float32)
                           + y_ref[...].astype(jnp.float32)) + b_ref[...])
        hmid = jnp.where(s >= 0, s, 0.2 * s)
        acc = (jnp.dot(hmid[0], w_ref[0], preferred_element_type=jnp.float32)
               + jnp.dot(hmid[1], w_ref[1], preferred_element_type=jnp.float32))
        yout_ref[...] = dinv * acc

    bn = _TC_GRID_BN
    hout = w2t.shape[-1]
    return pl.pallas_call(
        body,
        grid=(NP // bn,),
        in_specs=[
            pl.BlockSpec((NC, bn, 64), lambda i: (0, i, 0)),
            pl.BlockSpec((NC, bn, 64), lambda i: (0, i, 0)),
            pl.BlockSpec((bn, 1), lambda i: (i, 0)),
            pl.BlockSpec((NC, 1, 64), lambda i: (0, 0, 0)),
            pl.BlockSpec((NC, 64, hout), lambda i: (0, 0, 0)),
        ],
        out_specs=pl.BlockSpec((bn, hout), lambda i: (i, 0)),
        out_shape=jax.ShapeDtypeStruct((NP, hout), jnp.float32),
    )(p, y1t, dinv, b1t, w2t)


def _tc_mid(p0, p1, y, dinv, b, w, h, hout):
    def body(p0_ref, p1_ref, y_ref, dinv_ref, b_ref, w_ref, yout_ref):
        s = dinv_ref[...] * (p0_ref[...] + p1_ref[...] + y_ref[...]) + b_ref[...]
        hmid = jnp.where(s >= 0, s, 0.2 * s)
        yout_ref[...] = dinv_ref[...] * jnp.dot(hmid, w_ref[...],
                                                preferred_element_type=jnp.float32)

    bn = _TC_GRID_BN
    return pl.pallas_call(
        body,
        grid=(NP // bn,),
        in_specs=[
            pl.BlockSpec((bn, h), lambda i: (i, 0)),
            pl.BlockSpec((bn, h), lambda i: (i, 0)),
            pl.BlockSpec((bn, h), lambda i: (i, 0)),
            pl.BlockSpec((bn, 1), lambda i: (i, 0)),
            pl.BlockSpec((1, h), lambda i: (0, 0)),
            pl.BlockSpec((h, hout), lambda i: (0, 0)),
        ],
        out_specs=pl.BlockSpec((bn, hout), lambda i: (i, 0)),
        out_shape=jax.ShapeDtypeStruct((NP, hout), jnp.float32),
    )(p0, p1, y, dinv, b, w)


def _tc_final(p0, p1, y, dinv, b, h):
    def body(p0_ref, p1_ref, y_ref, dinv_ref, b_ref, out_ref):
        out_ref[...] = (dinv_ref[...] * (p0_ref[...] + p1_ref[...] + y_ref[...])
                        + b_ref[...])

    bn = _TC_GRID_BN
    return pl.pallas_call(
        body,
        grid=(NP // bn,),
        in_specs=[
            pl.BlockSpec((bn, h), lambda i: (i, 0)),
            pl.BlockSpec((bn, h), lambda i: (i, 0)),
            pl.BlockSpec((bn, h), lambda i: (i, 0)),
            pl.BlockSpec((bn, 1), lambda i: (i, 0)),
            pl.BlockSpec((1, h), lambda i: (0, 0)),
        ],
        out_specs=pl.BlockSpec((bn, h), lambda i: (i, 0)),
        out_shape=jax.ShapeDtypeStruct((NP, h), jnp.float32),
    )(p0, p1, y, dinv, b)


def kernel(x, edge_index, W1, b1, W2, b2, W3, b3):
    H1 = W1.shape[1]
    H2 = W2.shape[1]
    C = W3.shape[1]

    src = edge_index[0]
    dst = edge_index[1]
    pad_e = E_PAD - E
    pad_idx = jnp.full((pad_e,), N, dtype=jnp.int32)
    srcc = jnp.concatenate([src, pad_idx]).reshape(NCHUNKS, CH)
    dstc = jnp.concatenate([dst, pad_idx]).reshape(NCHUNKS, CH)

    xp = jnp.zeros((NP, D), jnp.float32).at[:N].set(x)
    ones1d = jnp.ones((CH,), jnp.float32)
    zeros1d = jnp.zeros((NP,), jnp.float32)
    zeros64 = jnp.zeros((NP, 64), jnp.bfloat16)
    zerosH2 = jnp.zeros((NP, H2), jnp.float32)
    zerosC = jnp.zeros((NP, C), jnp.float32)

    deg = _deg_call(dstc, zeros1d, ones1d)
    d0 = deg[0].reshape(NP, 1)
    d1 = deg[1].reshape(NP, 1)
    dinv, y1t = _tc_pre(xp, W1, d0, d1)

    pt = _prop_impl(y1t, srcc, dstc, zeros64, 64, True, jnp.bfloat16)
    y2 = _tc_mid1(pt, y1t, dinv, b1.reshape(NC, 1, 64), W2.reshape(NC, 64, H2))

    p = _prop_impl(y2, srcc, dstc, zerosH2, H2, False)
    y3 = _tc_mid(p[0], p[1], y2, dinv, b2.reshape(1, H2), W3, H2, C)

    p = _prop_impl(y3, srcc, dstc, zerosC, C, False)
    out = _tc_final(p[0], p[1], y3, dinv, b3.reshape(1, C), C)
    return out[:N]
